# baseline (speedup 1.0000x reference)
import jax
import jax.numpy as jnp
from jax.experimental import pallas as pl

_EPS = 1e-5


def _make_sweep_kernel(nlayers, write_y):
    def kern(*refs):
        x = refs[0][...]
        idx = 1
        for _ in range(nlayers):
            mv, g, be, w, b = refs[idx:idx + 5]
            idx += 5
            m = mv[0:1, :]
            v = mv[1:2, :]
            xn = g[...] * (x - m) * jax.lax.rsqrt(v + _EPS) + be[...]
            x = jnp.dot(xn, w[...], preferred_element_type=jnp.float32)
            x = jnp.maximum(x + b[...], 0.0)
        if write_y:
            y_ref, s_ref = refs[idx], refs[idx + 1]
            y_ref[...] = x
        else:
            s_ref = refs[idx]

        @pl.when(pl.program_id(0) == 0)
        def _():
            s_ref[...] = jnp.zeros_like(s_ref)

        upd = jnp.concatenate(
            [jnp.sum(x, axis=0, keepdims=True),
             jnp.sum(x * x, axis=0, keepdims=True)], axis=0)
        s_ref[...] = s_ref[...] + upd
    return kern


def _sweep(disp, layer_params, block_rows, write_y):
    s_rows = disp.shape[0]
    grid = s_rows // block_rows
    nlayers = len(layer_params)
    dout = layer_params[-1][3].shape[1]

    in_specs = [pl.BlockSpec((block_rows, disp.shape[1]), lambda i: (i, 0))]
    args = [disp]
    for mv, g, be, w, b in layer_params:
        din_j, dout_j = w.shape
        in_specs += [pl.BlockSpec((2, din_j), lambda i: (0, 0)),
                     pl.BlockSpec((1, din_j), lambda i: (0, 0)),
                     pl.BlockSpec((1, din_j), lambda i: (0, 0)),
                     pl.BlockSpec((din_j, dout_j), lambda i: (0, 0)),
                     pl.BlockSpec((1, dout_j), lambda i: (0, 0))]
        args += [mv, g.reshape(1, -1), be.reshape(1, -1), w, b.reshape(1, -1)]

    out_specs = []
    out_shape = []
    if write_y:
        out_specs.append(pl.BlockSpec((block_rows, dout), lambda i: (i, 0)))
        out_shape.append(jax.ShapeDtypeStruct((s_rows, dout), jnp.float32))
    out_specs.append(pl.BlockSpec((2, dout), lambda i: (0, 0)))
    out_shape.append(jax.ShapeDtypeStruct((2, dout), jnp.float32))

    res = pl.pallas_call(
        _make_sweep_kernel(nlayers, write_y),
        grid=(grid,),
        in_specs=in_specs,
        out_specs=out_specs,
        out_shape=out_shape,
    )(*args)
    if write_y:
        y, sums = res
    else:
        y, (sums,) = None, res
    mean = sums[0] / s_rows
    var = sums[1] / s_rows - mean * mean
    return y, jnp.stack([mean, var], axis=0)


def _bn_in_kernel(z, g, be):
    m = jnp.mean(z, axis=0, keepdims=True)
    v = jnp.mean((z - m) * (z - m), axis=0, keepdims=True)
    return g * (z - m) * jax.lax.rsqrt(v + _EPS) + be


def _out_kernel(x_ref, mv_ref, g3_ref, be3_ref,
                w0_ref, b0_ref, g0_ref, be0_ref,
                w1_ref, b1_ref, g1_ref, be1_ref, o_ref):
    raw = x_ref[...]
    m = mv_ref[0:1, :]
    v = mv_ref[1:2, :]
    feat = g3_ref[...] * (raw - m) * jax.lax.rsqrt(v + _EPS) + be3_ref[...]
    feat = jnp.where(jnp.isneginf(feat), 0.0, feat)
    z = jnp.dot(feat, w0_ref[...], preferred_element_type=jnp.float32)
    z = jnp.maximum(z + b0_ref[...], 0.0)
    z = _bn_in_kernel(z, g0_ref[...], be0_ref[...])
    z = jnp.dot(z, w1_ref[...], preferred_element_type=jnp.float32)
    z = jnp.maximum(z + b1_ref[...], 0.0)
    o_ref[...] = _bn_in_kernel(z, g1_ref[...], be1_ref[...])


def kernel(point_coordinates, keypoint_indices, set_indices,
           pW0, pb0, pg0, pbeta0, pW1, pb1, pg1, pbeta1,
           pW2, pb2, pg2, pbeta2, pW3, pb3, pg3, pbeta3,
           oW0, ob0, og0, obeta0, oW1, ob1, og1, obeta1):
    k_rows = keypoint_indices.shape[0]

    kp_coords = jnp.take(point_coordinates, keypoint_indices[:, 0], axis=0)
    src = jnp.take(point_coordinates, set_indices[:, 0], axis=0)
    dst = jnp.take(kp_coords, set_indices[:, 1], axis=0)
    disp = src - dst

    din0 = disp.shape[1]
    id_mv = jnp.stack([jnp.zeros((din0,), jnp.float32),
                       jnp.full((din0,), 1.0 - _EPS, jnp.float32)], axis=0)
    ones = jnp.ones((din0,), jnp.float32)
    zeros = jnp.zeros((din0,), jnp.float32)

    block = 5000
    l0 = (id_mv, ones, zeros, pW0, pb0)
    _, mv0 = _sweep(disp, [l0], block, write_y=False)
    l1 = (mv0, pg0, pbeta0, pW1, pb1)
    _, mv1 = _sweep(disp, [l0, l1], block, write_y=False)
    l2 = (mv1, pg1, pbeta1, pW2, pb2)
    _, mv2 = _sweep(disp, [l0, l1, l2], block, write_y=False)
    l3 = (mv2, pg2, pbeta2, pW3, pb3)
    y3, mv3 = _sweep(disp, [l0, l1, l2, l3], block, write_y=True)

    seg_raw = jax.ops.segment_max(y3, set_indices[:, 1], num_segments=k_rows)

    dout = oW1.shape[1]
    out = pl.pallas_call(
        _out_kernel,
        in_specs=[pl.BlockSpec(seg_raw.shape, lambda: (0, 0)),
                  pl.BlockSpec((2, 300), lambda: (0, 0)),
                  pl.BlockSpec((1, 300), lambda: (0, 0)),
                  pl.BlockSpec((1, 300), lambda: (0, 0)),
                  pl.BlockSpec(oW0.shape, lambda: (0, 0)),
                  pl.BlockSpec((1, 300), lambda: (0, 0)),
                  pl.BlockSpec((1, 300), lambda: (0, 0)),
                  pl.BlockSpec((1, 300), lambda: (0, 0)),
                  pl.BlockSpec(oW1.shape, lambda: (0, 0)),
                  pl.BlockSpec((1, dout), lambda: (0, 0)),
                  pl.BlockSpec((1, dout), lambda: (0, 0)),
                  pl.BlockSpec((1, dout), lambda: (0, 0))],
        out_specs=pl.BlockSpec((k_rows, dout), lambda: (0, 0)),
        out_shape=jax.ShapeDtypeStruct((k_rows, dout), jnp.float32),
    )(seg_raw, mv3, pg3.reshape(1, -1), pbeta3.reshape(1, -1),
      oW0, ob0.reshape(1, -1), og0.reshape(1, -1), obeta0.reshape(1, -1),
      oW1, ob1.reshape(1, -1), og1.reshape(1, -1), obeta1.reshape(1, -1))
    return out


# speedup vs baseline: 1.0400x; 1.0400x over previous
import jax
import jax.numpy as jnp
from jax.experimental import pallas as pl

_EPS = 1e-5


def _pt_layer_kernel(x_ref, w_ref, b_ref, g_ref, be_ref, mv_ref, y_ref, s_ref):
    i = pl.program_id(0)
    x = x_ref[...]
    m = mv_ref[0:1, :]
    v = mv_ref[1:2, :]
    xn = g_ref[...] * (x - m) * jax.lax.rsqrt(v + _EPS) + be_ref[...]
    y = jnp.dot(xn, w_ref[...], preferred_element_type=jnp.float32)
    y = jnp.maximum(y + b_ref[...], 0.0)
    y_ref[...] = y

    @pl.when(i == 0)
    def _():
        s_ref[...] = jnp.zeros_like(s_ref)

    upd = jnp.concatenate(
        [jnp.sum(y, axis=0, keepdims=True),
         jnp.sum(y * y, axis=0, keepdims=True)], axis=0)
    s_ref[...] = s_ref[...] + upd


def _pt_layer(x, w, b, g, be, mv, block_rows):
    s_rows, din = x.shape
    dout = w.shape[1]
    grid = s_rows // block_rows
    y, sums = pl.pallas_call(
        _pt_layer_kernel,
        grid=(grid,),
        in_specs=[
            pl.BlockSpec((block_rows, din), lambda i: (i, 0)),
            pl.BlockSpec((din, dout), lambda i: (0, 0)),
            pl.BlockSpec((1, dout), lambda i: (0, 0)),
            pl.BlockSpec((1, din), lambda i: (0, 0)),
            pl.BlockSpec((1, din), lambda i: (0, 0)),
            pl.BlockSpec((2, din), lambda i: (0, 0)),
        ],
        out_specs=[
            pl.BlockSpec((block_rows, dout), lambda i: (i, 0)),
            pl.BlockSpec((2, dout), lambda i: (0, 0)),
        ],
        out_shape=[
            jax.ShapeDtypeStruct((s_rows, dout), jnp.float32),
            jax.ShapeDtypeStruct((2, dout), jnp.float32),
        ],
    )(x, w, b.reshape(1, -1), g.reshape(1, -1), be.reshape(1, -1), mv)
    mean = sums[0] / s_rows
    var = sums[1] / s_rows - mean * mean
    return y, jnp.stack([mean, var], axis=0)


def _bn_in_kernel(z, g, be):
    m = jnp.mean(z, axis=0, keepdims=True)
    v = jnp.mean((z - m) * (z - m), axis=0, keepdims=True)
    return g * (z - m) * jax.lax.rsqrt(v + _EPS) + be


def _out_kernel(x_ref, mv_ref, g3_ref, be3_ref,
                w0_ref, b0_ref, g0_ref, be0_ref,
                w1_ref, b1_ref, g1_ref, be1_ref, o_ref):
    raw = x_ref[...]
    m = mv_ref[0:1, :]
    v = mv_ref[1:2, :]
    feat = g3_ref[...] * (raw - m) * jax.lax.rsqrt(v + _EPS) + be3_ref[...]
    feat = jnp.where(jnp.isneginf(feat), 0.0, feat)
    z = jnp.dot(feat, w0_ref[...], preferred_element_type=jnp.float32)
    z = jnp.maximum(z + b0_ref[...], 0.0)
    z = _bn_in_kernel(z, g0_ref[...], be0_ref[...])
    z = jnp.dot(z, w1_ref[...], preferred_element_type=jnp.float32)
    z = jnp.maximum(z + b1_ref[...], 0.0)
    o_ref[...] = _bn_in_kernel(z, g1_ref[...], be1_ref[...])


def kernel(point_coordinates, keypoint_indices, set_indices,
           pW0, pb0, pg0, pbeta0, pW1, pb1, pg1, pbeta1,
           pW2, pb2, pg2, pbeta2, pW3, pb3, pg3, pbeta3,
           oW0, ob0, og0, obeta0, oW1, ob1, og1, obeta1):
    s_rows = set_indices.shape[0]
    k_rows = keypoint_indices.shape[0]

    kp_coords = jnp.take(point_coordinates, keypoint_indices[:, 0], axis=0)
    src = jnp.take(point_coordinates, set_indices[:, 0], axis=0)
    dst = jnp.take(kp_coords, set_indices[:, 1], axis=0)
    disp = src - dst

    din0 = disp.shape[1]
    id_mv = jnp.stack([jnp.zeros((din0,), jnp.float32),
                       jnp.full((din0,), 1.0 - _EPS, jnp.float32)], axis=0)
    ones = jnp.ones((din0,), jnp.float32)
    zeros = jnp.zeros((din0,), jnp.float32)

    block = 5000
    x, mv = _pt_layer(disp, pW0, pb0, ones, zeros, id_mv, block)
    x, mv = _pt_layer(x, pW1, pb1, pg0, pbeta0, mv, block)
    x, mv = _pt_layer(x, pW2, pb2, pg1, pbeta1, mv, block)
    x, mv = _pt_layer(x, pW3, pb3, pg2, pbeta2, mv, block)

    seg_raw = jax.ops.segment_max(x, set_indices[:, 1], num_segments=k_rows)

    dout = oW1.shape[1]
    out = pl.pallas_call(
        _out_kernel,
        in_specs=[pl.BlockSpec(seg_raw.shape, lambda: (0, 0)),
                  pl.BlockSpec((2, 300), lambda: (0, 0)),
                  pl.BlockSpec((1, 300), lambda: (0, 0)),
                  pl.BlockSpec((1, 300), lambda: (0, 0)),
                  pl.BlockSpec(oW0.shape, lambda: (0, 0)),
                  pl.BlockSpec((1, 300), lambda: (0, 0)),
                  pl.BlockSpec((1, 300), lambda: (0, 0)),
                  pl.BlockSpec((1, 300), lambda: (0, 0)),
                  pl.BlockSpec(oW1.shape, lambda: (0, 0)),
                  pl.BlockSpec((1, dout), lambda: (0, 0)),
                  pl.BlockSpec((1, dout), lambda: (0, 0)),
                  pl.BlockSpec((1, dout), lambda: (0, 0))],
        out_specs=pl.BlockSpec((k_rows, dout), lambda: (0, 0)),
        out_shape=jax.ShapeDtypeStruct((k_rows, dout), jnp.float32),
    )(seg_raw, mv, pg3.reshape(1, -1), pbeta3.reshape(1, -1),
      oW0, ob0.reshape(1, -1), og0.reshape(1, -1), obeta0.reshape(1, -1),
      oW1, ob1.reshape(1, -1), og1.reshape(1, -1), obeta1.reshape(1, -1))
    return out
